# Initial kernel scaffold; baseline (speedup 1.0000x reference)
import functools

import jax
import jax.numpy as jnp
from jax import lax
from jax.experimental import pallas as pl
from jax.experimental.pallas import tpu as pltpu
from jax.experimental.pallas import tpu_sc as plsc


_NUM_WORKERS = 32
_GATHER_CHUNK = 1024


def _make_sc_gather(N, D):
    n_per_w = N // _NUM_WORKERS
    steps = n_per_w // _GATHER_CHUNK
    mesh = plsc.VectorSubcoreMesh(core_axis_name="c", subcore_axis_name="s")

    @functools.partial(
        pl.kernel,
        mesh=mesh,
        out_type=jax.ShapeDtypeStruct((N, D), jnp.float32),
        scratch_types=[
            pltpu.VMEM((_GATHER_CHUNK,), jnp.int32),
            pltpu.VMEM((_GATHER_CHUNK, D), jnp.float32),
            pltpu.SemaphoreType.DMA,
        ],
    )
    def gather_k(table_hbm, idx_hbm, out_hbm, idx_v, rows_v, sem):
        wid = lax.axis_index("s") * 2 + lax.axis_index("c")
        base = wid * n_per_w

        @pl.loop(0, steps)
        def _(i):
            off = base + i * _GATHER_CHUNK
            pltpu.sync_copy(idx_hbm.at[pl.ds(off, _GATHER_CHUNK)], idx_v)
            pltpu.async_copy(table_hbm.at[idx_v], rows_v, sem).wait()
            pltpu.sync_copy(rows_v, out_hbm.at[pl.ds(off, _GATHER_CHUNK)])

    return gather_k


_BLK = 2048


def _mlp_body(dx, lyb, tg, wb1, bb1, wb2, bb2, w1a, w1b, bt1, wt2, bt2, out):
    i = pl.program_id(0)
    f32 = jnp.float32
    x = jnp.dot(dx[...], wb1[...], preferred_element_type=f32) + bb1[...]
    x = jnp.dot(x, wb2[...], preferred_element_type=f32) + bb2[...]
    x = jnp.maximum(x, 0.0)
    h = (
        jnp.dot(x, w1a[...], preferred_element_type=f32)
        + jnp.dot(lyb[...], w1b[...], preferred_element_type=f32)
        + bt1[...]
    )
    s = jnp.dot(h, wt2[...], preferred_element_type=f32) + bt2[...]
    p = jax.nn.sigmoid(s)
    t = tg[...]
    log_p = jnp.maximum(jnp.log(p), -100.0)
    log_1mp = jnp.maximum(jnp.log(1.0 - p), -100.0)
    blk_sum = jnp.sum(t * log_p + (1.0 - t) * log_1mp)

    @pl.when(i == 0)
    def _():
        out[0, 0] = 0.0

    out[0, 0] += blk_sum


def _mlp_loss(dense_x, ly, target, W_bot1, b_bot1, W_bot2, b_bot2,
              W1a, W1b, b_top1, W_top2, b_top2):
    B = dense_x.shape[0]
    FD = ly.shape[1]
    grid = (B // _BLK,)
    full = lambda shape: pl.BlockSpec(shape, lambda i: (0, 0))
    out = pl.pallas_call(
        _mlp_body,
        grid=grid,
        in_specs=[
            pl.BlockSpec((_BLK, dense_x.shape[1]), lambda i: (i, 0)),
            pl.BlockSpec((_BLK, FD), lambda i: (i, 0)),
            pl.BlockSpec((_BLK, 1), lambda i: (i, 0)),
            full(W_bot1.shape),
            full(b_bot1.shape),
            full(W_bot2.shape),
            full(b_bot2.shape),
            full(W1a.shape),
            full(W1b.shape),
            full(b_top1.shape),
            full(W_top2.shape),
            full(b_top2.shape),
        ],
        out_specs=pl.BlockSpec(memory_space=pltpu.SMEM),
        out_shape=jax.ShapeDtypeStruct((1, 1), jnp.float32),
    )(dense_x, ly, target, W_bot1, b_bot1, W_bot2, b_bot2,
      W1a, W1b, b_top1, W_top2, b_top2)
    return out


def kernel(dense_x, ls_i, target, W_bot1, b_bot1, W_bot2, b_bot2, emb,
           W_top1, b_top1, W_top2, b_top2):
    F, V, D = emb.shape
    B = dense_x.shape[0]
    N = F * B

    table = emb.reshape(F * V, D)
    idx = (ls_i.T + (jnp.arange(F, dtype=jnp.int32) * V)[None, :]).reshape(N)

    rows = _make_sc_gather(N, D)(table, idx)
    ly = rows.reshape(B, F * D)

    loss_sum = _mlp_loss(
        dense_x, ly, target,
        W_bot1, b_bot1.reshape(1, -1), W_bot2, b_bot2.reshape(1, -1),
        W_top1[:D], W_top1[D:], b_top1.reshape(1, -1),
        W_top2, b_top2.reshape(1, 1),
    )
    return -loss_sum[0, 0] / B


# speedup vs baseline: 5.8183x; 5.8183x over previous
import functools

import jax
import jax.numpy as jnp
from jax import lax
from jax.experimental import pallas as pl
from jax.experimental.pallas import tpu as pltpu
from jax.experimental.pallas import tpu_sc as plsc


_NUM_WORKERS = 32
_GATHER_CHUNK = 1024


def _make_sc_gather(N, D):
    n_per_w = N // _NUM_WORKERS
    steps = n_per_w // _GATHER_CHUNK
    mesh = plsc.VectorSubcoreMesh(core_axis_name="c", subcore_axis_name="s")

    @functools.partial(
        pl.kernel,
        mesh=mesh,
        out_type=jax.ShapeDtypeStruct((N, D), jnp.float32),
        compiler_params=pltpu.CompilerParams(use_tc_tiling_on_sc=False),
        scratch_types=[
            pltpu.VMEM((_GATHER_CHUNK,), jnp.int32),
            pltpu.VMEM((_GATHER_CHUNK, D), jnp.float32),
            pltpu.SemaphoreType.DMA,
        ],
    )
    def gather_k(table_hbm, idx_hbm, out_hbm, idx_v, rows_v, sem):
        wid = lax.axis_index("s") * 2 + lax.axis_index("c")
        base = wid * n_per_w

        @pl.loop(0, steps)
        def _(i):
            off = base + i * _GATHER_CHUNK
            pltpu.sync_copy(idx_hbm.at[pl.ds(off, _GATHER_CHUNK)], idx_v)
            pltpu.async_copy(table_hbm.at[idx_v], rows_v, sem).wait()
            pltpu.sync_copy(rows_v, out_hbm.at[pl.ds(off, _GATHER_CHUNK)])

    return gather_k


_BLK = 2048


def _mlp_body(dx, lyb, tg, wb1, bb1, wb2, bb2, w1a, w1b, bt1, wt2, bt2, out):
    i = pl.program_id(0)
    f32 = jnp.float32
    x = jnp.dot(dx[...], wb1[...], preferred_element_type=f32) + bb1[...]
    x = jnp.dot(x, wb2[...], preferred_element_type=f32) + bb2[...]
    x = jnp.maximum(x, 0.0)
    h = (
        jnp.dot(x, w1a[...], preferred_element_type=f32)
        + jnp.dot(lyb[...], w1b[...], preferred_element_type=f32)
        + bt1[...]
    )
    s = jnp.dot(h, wt2[...], preferred_element_type=f32) + bt2[...]
    p = jax.nn.sigmoid(s)
    t = tg[...]
    log_p = jnp.maximum(jnp.log(p), -100.0)
    log_1mp = jnp.maximum(jnp.log(1.0 - p), -100.0)
    blk_sum = jnp.sum(t * log_p + (1.0 - t) * log_1mp)

    @pl.when(i == 0)
    def _():
        out[0, 0] = 0.0

    out[0, 0] += blk_sum


def _mlp_loss(dense_x, ly, target, W_bot1, b_bot1, W_bot2, b_bot2,
              W1a, W1b, b_top1, W_top2, b_top2):
    B = dense_x.shape[0]
    FD = ly.shape[1]
    grid = (B // _BLK,)
    full = lambda shape: pl.BlockSpec(shape, lambda i: (0, 0))
    out = pl.pallas_call(
        _mlp_body,
        grid=grid,
        in_specs=[
            pl.BlockSpec((_BLK, dense_x.shape[1]), lambda i: (i, 0)),
            pl.BlockSpec((_BLK, FD), lambda i: (i, 0)),
            pl.BlockSpec((_BLK, 1), lambda i: (i, 0)),
            full(W_bot1.shape),
            full(b_bot1.shape),
            full(W_bot2.shape),
            full(b_bot2.shape),
            full(W1a.shape),
            full(W1b.shape),
            full(b_top1.shape),
            full(W_top2.shape),
            full(b_top2.shape),
        ],
        out_specs=pl.BlockSpec(memory_space=pltpu.SMEM),
        out_shape=jax.ShapeDtypeStruct((1, 1), jnp.float32),
    )(dense_x, ly, target, W_bot1, b_bot1, W_bot2, b_bot2,
      W1a, W1b, b_top1, W_top2, b_top2)
    return out


def kernel(dense_x, ls_i, target, W_bot1, b_bot1, W_bot2, b_bot2, emb,
           W_top1, b_top1, W_top2, b_top2):
    F, V, D = emb.shape
    B = dense_x.shape[0]
    N = F * B

    table = emb.reshape(F * V, D)
    idx = (ls_i.T + (jnp.arange(F, dtype=jnp.int32) * V)[None, :]).reshape(N)

    rows = _make_sc_gather(N, D)(table, idx)
    ly = rows.reshape(B, F * D)

    loss_sum = _mlp_loss(
        dense_x, ly, target,
        W_bot1, b_bot1.reshape(1, -1), W_bot2, b_bot2.reshape(1, -1),
        W_top1[:D], W_top1[D:], b_top1.reshape(1, -1),
        W_top2, b_top2.reshape(1, 1),
    )
    return -loss_sum[0, 0] / B
